# baseline (speedup 1.0000x reference)
import functools

import jax
import jax.numpy as jnp
from jax import lax
from jax.experimental import pallas as pl
from jax.experimental.pallas import tpu as pltpu
from jax.experimental.pallas import tpu_sc as plsc

NC = 2
NS = 16
NW = NC * NS
LANES = 16
BBLK = 128
NBUF = 4
DPAD = 128


def _gather_body(seq, d, w_hbm, xt_hbm, out_hbm,
                 xv, pairbuf, outblk, sem_g, sem_o):
    wid = lax.axis_index("s") * NC + lax.axis_index("c")
    b0 = wid * BBLK

    pltpu.sync_copy(xt_hbm.at[:, pl.ds(b0, BBLK)], xv)

    def fire_gather(l, b):
        pltpu.async_copy(w_hbm.at[xv.at[l, :]], pairbuf.at[b], sem_g.at[b])

    def wait_gather(b):
        pltpu.make_async_copy(
            w_hbm.at[pl.ds(0, BBLK)], pairbuf.at[b], sem_g.at[b]).wait()

    def wait_out(b):
        pltpu.make_async_copy(
            outblk.at[b], out_hbm.at[0, :, pl.ds(0, BBLK)],
            sem_o.at[b]).wait()

    def compact_transpose(b):
        iota = lax.iota(jnp.int32, LANES)
        rowvecs = [iota + dg * LANES for dg in range(2)]

        @plsc.parallel_loop(0, BBLK, unroll=16)
        def _t(bb):
            col = jnp.full((LANES,), 0, jnp.int32) + bb
            for dg in range(2):
                v = pairbuf[b, bb, pl.ds(dg * LANES, LANES)]
                plsc.store_scatter(outblk.at[b], [rowvecs[dg], col], v)
            for j in range(2):
                k = bb * 2 + j
                dd = 32 + lax.shift_right_logical(k, 3)
                bg16 = lax.bitwise_and(k, 7) * LANES
                rows = iota + bg16
                cvec = jnp.full((LANES,), 0, jnp.int32) + dd
                v = plsc.load_gather(pairbuf.at[b], [rows, cvec])
                outblk[b, dd, pl.ds(bg16, LANES)] = v

    fire_gather(0, 0)

    @pl.loop(0, seq // NBUF)
    def _g(g):
        for b in range(NBUF):
            l = g * NBUF + b
            nb = (b + 1) % NBUF
            wait_gather(b)

            @pl.when(l + 1 < seq)
            def _():
                fire_gather(l + 1, nb)

            compact_transpose(b)

            @pl.when(g > 0)
            def _():
                wait_out(b)

            pltpu.async_copy(
                outblk.at[b], out_hbm.at[l, :, pl.ds(b0, BBLK)],
                sem_o.at[b])

    for b in range(NBUF):
        wait_out(b)


def _embedding_gather(x, W):
    V, D = W.shape
    B, S = x.shape
    assert B == NW * BBLK and S % NBUF == 0

    W2x = jnp.pad(W, ((0, 0), (0, DPAD - D)))
    xt = jnp.transpose(x)

    mesh = plsc.VectorSubcoreMesh(
        core_axis_name="c", subcore_axis_name="s",
        num_cores=NC, num_subcores=NS)

    body = functools.partial(_gather_body, S, D)
    out_phys = pl.kernel(
        body,
        out_type=jax.ShapeDtypeStruct((S, D, B), W.dtype),
        mesh=mesh,
        scratch_types=[
            pltpu.VMEM((S, BBLK), jnp.int32),
            pltpu.VMEM((NBUF, BBLK, DPAD), W.dtype),
            pltpu.VMEM((NBUF, D, BBLK), W.dtype),
            pltpu.SemaphoreType.DMA((NBUF,)),
            pltpu.SemaphoreType.DMA((NBUF,)),
        ],
        compiler_params=pltpu.CompilerParams(
            use_tc_tiling_on_sc=True, needs_layout_passes=False),
    )(W2x, xt)
    return jnp.transpose(out_phys, (2, 0, 1))


def kernel(x, target, text_inputs, W):
    return _embedding_gather(x, W)


# speedup vs baseline: 1.0172x; 1.0172x over previous
import functools

import jax
import jax.numpy as jnp
from jax import lax
from jax.experimental import pallas as pl
from jax.experimental.pallas import tpu as pltpu
from jax.experimental.pallas import tpu_sc as plsc

NC = 2
NS = 16
NW = NC * NS
LANES = 16
BBLK = 128
NBUF = 4
DPAD = 128


def _gather_body(seq, d, w_hbm, xt_hbm, out_hbm,
                 xv, pairbuf, outblk, sem_g, sem_o):
    wid = lax.axis_index("s") * NC + lax.axis_index("c")
    b0 = wid * BBLK

    pltpu.sync_copy(xt_hbm.at[:, pl.ds(b0, BBLK)], xv)

    def fire_gather(l, b):
        pltpu.async_copy(w_hbm.at[xv.at[l, :]], pairbuf.at[b], sem_g.at[b])

    def wait_gather(b):
        pltpu.make_async_copy(
            w_hbm.at[pl.ds(0, BBLK)], pairbuf.at[b], sem_g.at[b]).wait()

    def wait_out(b):
        pltpu.make_async_copy(
            outblk.at[b], out_hbm.at[0, :, pl.ds(0, BBLK)],
            sem_o.at[b]).wait()

    def compact_transpose(b):
        iota = lax.iota(jnp.int32, LANES)
        rowvecs = [iota + dg * LANES for dg in range(2)]

        @plsc.parallel_loop(0, BBLK, unroll=8)
        def _t(bb):
            col = jnp.full((LANES,), 0, jnp.int32) + bb
            for dg in range(2):
                v = pairbuf[b, bb, pl.ds(dg * LANES, LANES)]
                plsc.store_scatter(outblk.at[b], [rowvecs[dg], col], v)
            for j in range(2):
                k = bb * 2 + j
                dd = 32 + lax.shift_right_logical(k, 3)
                bg16 = lax.bitwise_and(k, 7) * LANES
                rows = iota + bg16
                cvec = jnp.full((LANES,), 0, jnp.int32) + dd
                v = plsc.load_gather(pairbuf.at[b], [rows, cvec])
                outblk[b, dd, pl.ds(bg16, LANES)] = v

    fire_gather(0, 0)

    @pl.loop(0, seq // NBUF)
    def _g(g):
        for b in range(NBUF):
            l = g * NBUF + b
            nb = (b + 1) % NBUF
            wait_gather(b)

            @pl.when(l + 1 < seq)
            def _():
                fire_gather(l + 1, nb)

            compact_transpose(b)

            @pl.when(g > 0)
            def _():
                wait_out(b)

            pltpu.async_copy(
                outblk.at[b], out_hbm.at[l, :, pl.ds(b0, BBLK)],
                sem_o.at[b])

    for b in range(NBUF):
        wait_out(b)


def _embedding_gather(x, W):
    V, D = W.shape
    B, S = x.shape
    assert B == NW * BBLK and S % NBUF == 0

    W2x = jnp.pad(W, ((0, 0), (0, DPAD - D)))
    xt = jnp.transpose(x)

    mesh = plsc.VectorSubcoreMesh(
        core_axis_name="c", subcore_axis_name="s",
        num_cores=NC, num_subcores=NS)

    body = functools.partial(_gather_body, S, D)
    out_phys = pl.kernel(
        body,
        out_type=jax.ShapeDtypeStruct((S, D, B), W.dtype),
        mesh=mesh,
        scratch_types=[
            pltpu.VMEM((S, BBLK), jnp.int32),
            pltpu.VMEM((NBUF, BBLK, DPAD), W.dtype),
            pltpu.VMEM((NBUF, D, BBLK), W.dtype),
            pltpu.SemaphoreType.DMA((NBUF,)),
            pltpu.SemaphoreType.DMA((NBUF,)),
        ],
        compiler_params=pltpu.CompilerParams(
            use_tc_tiling_on_sc=True, needs_layout_passes=False),
    )(W2x, xt)
    return jnp.transpose(out_phys, (2, 0, 1))


def kernel(x, target, text_inputs, W):
    return _embedding_gather(x, W)
